# baseline (speedup 1.0000x reference)
import jax
import jax.numpy as jnp
from jax import lax
from jax.experimental import pallas as pl
from jax.experimental.pallas import tpu as pltpu

B, T, V = 16, 512, 2048
D = 128
LEVELS = 3
CHUNK = 128
HIDDEN = CHUNK * LEVELS
CONV = 10
OUT_DIM = 128
GATE_REST = 4 * LEVELS * CHUNK
XH = D + HIDDEN
PAD = 16
TB = 64
NT = T // TB


def _emb_body(bd_ref, emb_ref, x_ref, cnt_ref):
    bd = bd_ref[0]
    xf = (bd == 1).astype(jnp.bfloat16)
    y = jnp.dot(xf, emb_ref[...], preferred_element_type=jnp.float32)
    x_ref[0] = y
    m = jnp.max(jnp.abs(y), axis=1, keepdims=True)
    cnt = jnp.sum((m > 0.0).astype(jnp.float32))
    cnt_ref[...] = jnp.full((1, 8, 128), cnt, jnp.float32)


def _x1_body(x_ref, wkm_ref, wkr_ref, bm_ref, br_ref, x1m_ref, x1r_ref):
    xf = x_ref[...].reshape(TB * B, D).astype(jnp.bfloat16)
    x1m = (jnp.dot(xf, wkm_ref[...], preferred_element_type=jnp.float32)
           + bm_ref[...])
    x1r = (jnp.dot(xf, wkr_ref[...], preferred_element_type=jnp.float32)
           + br_ref[...])
    x1m_ref[...] = x1m.reshape(TB, B, 128)
    x1r_ref[...] = x1r.astype(jnp.bfloat16).reshape(TB, B, GATE_REST)


def _rec_body(x1m_ref, x1r_ref, wm_ref, wrr_ref, hseq_ref, dis_ref,
              c_ref, h_ref, d_ref):
    c_ref[...] = jnp.zeros((B, HIDDEN), jnp.float32)
    h_ref[...] = jnp.zeros((B, HIDDEN), jnp.float32)
    d_ref[...] = jnp.zeros((B, 128), jnp.float32)
    hseq_ref[0:PAD] = jnp.zeros((PAD, B, HIDDEN), jnp.bfloat16)

    def one_step(t, c, h, dbuf):
        hb = h.astype(jnp.bfloat16)
        x1r = x1r_ref[t]
        grp = []
        for g in range(4):
            xg = (jnp.dot(hb, wrr_ref[:, g * HIDDEN:(g + 1) * HIDDEN],
                          preferred_element_type=jnp.float32)
                  + x1r[:, g * HIDDEN:(g + 1) * HIDDEN].astype(jnp.float32))
            grp.append(xg)
        xom = x1m_ref[t] + jnp.dot(hb, wm_ref[...],
                                   preferred_element_type=jnp.float32)

        f_in = xom[:, 0:3]
        i_in = xom[:, 3:6]
        fe = jnp.exp(f_in - jnp.max(f_in, axis=1, keepdims=True))
        fp = fe / jnp.sum(fe, axis=1, keepdims=True)
        p0, p1, p2 = fp[:, 0:1], fp[:, 1:2], fp[:, 2:3]
        fm = (p0, p0 + p1, (p0 + p1) + p2)
        ie = jnp.exp(i_in - jnp.max(i_in, axis=1, keepdims=True))
        ip = ie / jnp.sum(ie, axis=1, keepdims=True)
        q0, q1, q2 = ip[:, 0:1], ip[:, 1:2], ip[:, 2:3]
        im = ((q2 + q1) + q0, q2 + q1, q2)

        c_parts = []
        h_parts = []
        for l in range(LEVELS):
            fg = jax.nn.sigmoid(grp[0][:, l * CHUNK:(l + 1) * CHUNK])
            ig = jax.nn.sigmoid(grp[1][:, l * CHUNK:(l + 1) * CHUNK])
            og = jax.nn.sigmoid(grp[2][:, l * CHUNK:(l + 1) * CHUNK])
            ci = jnp.tanh(grp[3][:, l * CHUNK:(l + 1) * CHUNK])
            cl = c[:, l * CHUNK:(l + 1) * CHUNK]
            ov = fm[l] * im[l]
            c3 = cl * (ov * (fg - 1.0) + fm[l]) + ci * (ov * (ig - 1.0)
                                                        + im[l])
            h_parts.append(og * jnp.tanh(c3))
            c_parts.append(c3)
        c_new = jnp.concatenate(c_parts, axis=1)
        h_new = jnp.concatenate(h_parts, axis=1)

        cur_dis = 1.0 - (fm[0] + fm[1] + fm[2]) * (1.0 / 3.0)
        dnew = jnp.concatenate(
            [dbuf[:, 1:10], cur_dis, dbuf[:, 10:128]], axis=1)

        run = dnew[:, 0:1]
        cs = [run]
        for k in range(1, CONV):
            run = run + dnew[:, k:k + 1]
            cs.append(run)
        mx = cs[0]
        for k in range(1, CONV):
            mx = jnp.maximum(mx, cs[k])
        es = [jnp.exp(v - mx) for v in cs]
        tot = es[0]
        for k in range(1, CONV):
            tot = tot + es[k]
        inv = 1.0 / tot
        dn = jnp.concatenate([e * inv for e in es]
                             + [jnp.zeros((B, 128 - CONV), jnp.float32)],
                             axis=1)

        hseq_ref[PAD + t] = h_new.astype(jnp.bfloat16)
        dis_ref[t] = dn
        return c_new, h_new, dnew

    def step2(i, _):
        c = c_ref[...]
        h = h_ref[...]
        dbuf = d_ref[...]
        t = 2 * i
        c, h, dbuf = one_step(t, c, h, dbuf)
        c, h, dbuf = one_step(t + 1, c, h, dbuf)
        c_ref[...] = c
        h_ref[...] = h
        d_ref[...] = dbuf
        return 0

    jax.lax.fori_loop(0, T // 2, step2, 0)


def _win_body(hseq_ref, dis_ref, li_ref, wc_ref, sw_ref, sb_ref, rw_ref,
              rb_ref, cb_ref, fw_ref, fb_ref, out_ref, acc_ref):
    tb = pl.program_id(0)
    t0 = tb * TB

    @pl.when(tb == 0)
    def _init():
        acc_ref[...] = jnp.zeros((B, HIDDEN), jnp.float32)

    theme = None
    conv = None
    for k in range(CONV):
        hk = hseq_ref[pl.ds(t0 + PAD - (CONV - 1) + k, TB)]
        dk = dis_ref[pl.ds(t0, TB), :, k:k + 1]
        shk = (hk.astype(jnp.float32).reshape(TB * B, HIDDEN)
               * dk.reshape(TB * B, 1))
        theme = shk if theme is None else theme + shk
        pk = jnp.dot(shk.astype(jnp.bfloat16),
                     wc_ref[k * HIDDEN:(k + 1) * HIDDEN],
                     preferred_element_type=jnp.float32)
        conv = pk if conv is None else conv + pk
    s1 = jnp.maximum(
        jnp.dot((theme * (1.0 / CONV)).astype(jnp.bfloat16), sw_ref[...],
                preferred_element_type=jnp.float32) + sb_ref[...], 0.0)
    s2 = jax.nn.sigmoid(
        jnp.dot(s1.astype(jnp.bfloat16), rw_ref[...],
                preferred_element_type=jnp.float32) + rb_ref[...])
    h_t = s2 * (conv + cb_ref[...])
    hcen = hseq_ref[pl.ds(t0 + PAD, TB)].astype(jnp.float32)
    rnn = h_t.reshape(TB, B, HIDDEN) + hcen

    tvec = t0 + lax.broadcasted_iota(jnp.int32, (TB, B, 1), 0)
    m = (tvec == li_ref[...].reshape(1, B, 1)).astype(jnp.float32)
    acc_ref[...] += jnp.sum(rnn * m, axis=0)

    @pl.when(tb == NT - 1)
    def _fin():
        out_ref[...] = (jnp.dot(acc_ref[...], fw_ref[...],
                                preferred_element_type=jnp.float32)
                        + fb_ref[...])


@jax.jit
def kernel(batchdata, emb_table, kernel_W, kernel_b, rec_W, rec_b, scale_W,
           scale_b, rescale_W, rescale_b, conv_W, conv_b, fc_W, fc_b):
    x, cnt = pl.pallas_call(
        _emb_body,
        grid=(B,),
        in_specs=[
            pl.BlockSpec((1, T, V), lambda b: (b, 0, 0)),
            pl.BlockSpec((V, D), lambda b: (0, 0)),
        ],
        out_specs=[
            pl.BlockSpec((1, T, D), lambda b: (b, 0, 0)),
            pl.BlockSpec((1, 8, 128), lambda b: (b, 0, 0)),
        ],
        out_shape=[
            jax.ShapeDtypeStruct((B, T, D), jnp.float32),
            jax.ShapeDtypeStruct((B, 8, 128), jnp.float32),
        ],
    )(batchdata, emb_table.astype(jnp.bfloat16))

    xT = jnp.transpose(x, (1, 0, 2))
    li = jnp.clip(cnt[:, 0, 0].astype(jnp.int32) - 1, 0, T - 1).reshape(B, 1)

    wkm = jnp.zeros((D, 128), jnp.float32).at[:, 0:6].set(kernel_W[0:6, 0:D].T)
    wkr = kernel_W[6:, 0:D].T
    wm = jnp.zeros((HIDDEN, 128), jnp.float32).at[:, 0:6].set(
        rec_W[0:6, 0:HIDDEN].T)
    wrr = rec_W[6:, 0:HIDDEN].T
    bias_full = kernel_b + kernel_W[:, D] + rec_b + rec_W[:, HIDDEN]
    bm = jnp.zeros((1, 128), jnp.float32).at[0, 0:6].set(bias_full[0:6])
    br = bias_full[6:].reshape(1, GATE_REST)
    wc = jnp.transpose(conv_W, (2, 1, 0)).reshape(CONV * HIDDEN, HIDDEN)
    sw = scale_W.T
    sb = scale_b.reshape(1, -1)
    rw = rescale_W.T
    rb = rescale_b.reshape(1, -1)
    cb = conv_b.reshape(1, -1)
    fw = fc_W.T
    fb = fc_b.reshape(1, -1)

    bf = jnp.bfloat16
    full = lambda shape: pl.BlockSpec(shape, lambda: tuple(0 for _ in shape))
    gfull = lambda shape: pl.BlockSpec(shape,
                                       lambda i: tuple(0 for _ in shape))
    x1m, x1r = pl.pallas_call(
        _x1_body,
        grid=(NT,),
        in_specs=[
            pl.BlockSpec((TB, B, D), lambda i: (i, 0, 0)),
            gfull((D, 128)),
            gfull((D, GATE_REST)),
            gfull((1, 128)),
            gfull((1, GATE_REST)),
        ],
        out_specs=[
            pl.BlockSpec((TB, B, 128), lambda i: (i, 0, 0)),
            pl.BlockSpec((TB, B, GATE_REST), lambda i: (i, 0, 0)),
        ],
        out_shape=[
            jax.ShapeDtypeStruct((T, B, 128), jnp.float32),
            jax.ShapeDtypeStruct((T, B, GATE_REST), jnp.bfloat16),
        ],
    )(xT, wkm.astype(bf), wkr.astype(bf), bm, br)

    rec_args = (x1m, x1r, wm.astype(bf), wrr.astype(bf))
    hseq, dis = pl.pallas_call(
        _rec_body,
        in_specs=[full(a.shape) for a in rec_args],
        out_specs=[full((PAD + T, B, HIDDEN)), full((T, B, 128))],
        out_shape=[
            jax.ShapeDtypeStruct((PAD + T, B, HIDDEN), jnp.bfloat16),
            jax.ShapeDtypeStruct((T, B, 128), jnp.float32),
        ],
        scratch_shapes=[
            pltpu.VMEM((B, HIDDEN), jnp.float32),
            pltpu.VMEM((B, HIDDEN), jnp.float32),
            pltpu.VMEM((B, 128), jnp.float32),
        ],
    )(*rec_args)

    win_args = (hseq, dis, li, wc.astype(bf), sw.astype(bf), sb,
                rw.astype(bf), rb, cb, fw, fb)
    logits = pl.pallas_call(
        _win_body,
        grid=(NT,),
        in_specs=[gfull(a.shape) for a in win_args],
        out_specs=gfull((B, OUT_DIM)),
        out_shape=jax.ShapeDtypeStruct((B, OUT_DIM), jnp.float32),
        scratch_shapes=[pltpu.VMEM((B, HIDDEN), jnp.float32)],
    )(*win_args)
    return logits


# speedup vs baseline: 1.0262x; 1.0262x over previous
import jax
import jax.numpy as jnp
from jax import lax
from jax.experimental import pallas as pl
from jax.experimental.pallas import tpu as pltpu

B, T, V = 16, 512, 2048
D = 128
LEVELS = 3
CHUNK = 128
HIDDEN = CHUNK * LEVELS
CONV = 10
OUT_DIM = 128
GATE_REST = 4 * LEVELS * CHUNK
XH = D + HIDDEN
PAD = 16
TB = 64
NT = T // TB


def _emb_body(bd_ref, emb_ref, x_ref, cnt_ref):
    bd = bd_ref[0]
    xf = (bd == 1).astype(jnp.bfloat16)
    y = jnp.dot(xf, emb_ref[...], preferred_element_type=jnp.float32)
    x_ref[0] = y
    m = jnp.max(jnp.abs(y), axis=1, keepdims=True)
    cnt = jnp.sum((m > 0.0).astype(jnp.float32))
    cnt_ref[...] = jnp.full((1, 8, 128), cnt, jnp.float32)


def _x1_body(x_ref, wkm_ref, wkr_ref, bm_ref, br_ref, x1m_ref, x1r_ref):
    xf = x_ref[...].reshape(TB * B, D).astype(jnp.bfloat16)
    x1m = (jnp.dot(xf, wkm_ref[...], preferred_element_type=jnp.float32)
           + bm_ref[...])
    x1r = (jnp.dot(xf, wkr_ref[...], preferred_element_type=jnp.float32)
           + br_ref[...])
    x1m_ref[...] = x1m.reshape(TB, B, 128)
    x1r_ref[...] = x1r.astype(jnp.bfloat16).reshape(TB, B, GATE_REST)


def _rec_body(x1m_ref, x1r_ref, wm_ref, wrr_ref, li_ref, wc_ref, sw_ref,
              sb_ref, rw_ref, rb_ref, cb_ref, fw_ref, fb_ref, out_ref,
              c_ref, h_ref, d_ref, hseq_ref, dis_ref, acc_ref):
    wm = wm_ref[...]
    wrr = wrr_ref[...]
    c_ref[...] = jnp.zeros((B, HIDDEN), jnp.float32)
    h_ref[...] = jnp.zeros((B, HIDDEN), jnp.float32)
    d_ref[...] = jnp.zeros((B, 128), jnp.float32)
    hseq_ref[0:PAD] = jnp.zeros((PAD, B, HIDDEN), jnp.bfloat16)

    def one_step(t, c, h, dbuf):
        hb = h.astype(jnp.bfloat16)
        xor_ = (jnp.dot(hb, wrr, preferred_element_type=jnp.float32)
                + x1r_ref[t].astype(jnp.float32))
        grp = [xor_[:, g * HIDDEN:(g + 1) * HIDDEN] for g in range(4)]
        xom = x1m_ref[t] + jnp.dot(hb, wm,
                                   preferred_element_type=jnp.float32)

        f_in = xom[:, 0:3]
        i_in = xom[:, 3:6]
        fe = jnp.exp(f_in - jnp.max(f_in, axis=1, keepdims=True))
        fp = fe / jnp.sum(fe, axis=1, keepdims=True)
        p0, p1, p2 = fp[:, 0:1], fp[:, 1:2], fp[:, 2:3]
        fm = (p0, p0 + p1, (p0 + p1) + p2)
        ie = jnp.exp(i_in - jnp.max(i_in, axis=1, keepdims=True))
        ip = ie / jnp.sum(ie, axis=1, keepdims=True)
        q0, q1, q2 = ip[:, 0:1], ip[:, 1:2], ip[:, 2:3]
        im = ((q2 + q1) + q0, q2 + q1, q2)

        c_parts = []
        h_parts = []
        for l in range(LEVELS):
            fg = jax.nn.sigmoid(grp[0][:, l * CHUNK:(l + 1) * CHUNK])
            ig = jax.nn.sigmoid(grp[1][:, l * CHUNK:(l + 1) * CHUNK])
            og = jax.nn.sigmoid(grp[2][:, l * CHUNK:(l + 1) * CHUNK])
            ci = jnp.tanh(grp[3][:, l * CHUNK:(l + 1) * CHUNK])
            cl = c[:, l * CHUNK:(l + 1) * CHUNK]
            ov = fm[l] * im[l]
            c3 = cl * (ov * (fg - 1.0) + fm[l]) + ci * (ov * (ig - 1.0)
                                                        + im[l])
            h_parts.append(og * jnp.tanh(c3))
            c_parts.append(c3)
        c_new = jnp.concatenate(c_parts, axis=1)
        h_new = jnp.concatenate(h_parts, axis=1)

        cur_dis = 1.0 - (fm[0] + fm[1] + fm[2]) * (1.0 / 3.0)
        dnew = jnp.concatenate(
            [dbuf[:, 1:10], cur_dis, dbuf[:, 10:128]], axis=1)

        run = dnew[:, 0:1]
        cs = [run]
        for k in range(1, CONV):
            run = run + dnew[:, k:k + 1]
            cs.append(run)
        mx = cs[0]
        for k in range(1, CONV):
            mx = jnp.maximum(mx, cs[k])
        es = [jnp.exp(v - mx) for v in cs]
        tot = es[0]
        for k in range(1, CONV):
            tot = tot + es[k]
        inv = 1.0 / tot
        dn = jnp.concatenate([e * inv for e in es]
                             + [jnp.zeros((B, 128 - CONV), jnp.float32)],
                             axis=1)

        hseq_ref[PAD + t] = h_new.astype(jnp.bfloat16)
        dis_ref[t] = dn
        return c_new, h_new, dnew

    UNROLL = 2

    def stepn(i, _):
        c = c_ref[...]
        h = h_ref[...]
        dbuf = d_ref[...]
        t = UNROLL * i
        for u in range(UNROLL):
            c, h, dbuf = one_step(t + u, c, h, dbuf)
        c_ref[...] = c
        h_ref[...] = h
        d_ref[...] = dbuf
        return 0

    jax.lax.fori_loop(0, T // UNROLL, stepn, 0)

    li = li_ref[...]
    acc_ref[...] = jnp.zeros((B, HIDDEN), jnp.float32)

    def win_block(tb, _):
        t0 = tb * TB
        theme = None
        conv = None
        for k in range(CONV):
            hk = hseq_ref[pl.ds(t0 + PAD - (CONV - 1) + k, TB)]
            dk = dis_ref[pl.ds(t0, TB), :, k:k + 1]
            shk = (hk.astype(jnp.float32).reshape(TB * B, HIDDEN)
                   * dk.reshape(TB * B, 1))
            theme = shk if theme is None else theme + shk
            pk = jnp.dot(shk.astype(jnp.bfloat16),
                         wc_ref[k * HIDDEN:(k + 1) * HIDDEN],
                         preferred_element_type=jnp.float32)
            conv = pk if conv is None else conv + pk
        s1 = jnp.maximum(
            jnp.dot((theme * (1.0 / CONV)).astype(jnp.bfloat16), sw_ref[...],
                    preferred_element_type=jnp.float32) + sb_ref[...], 0.0)
        s2 = jax.nn.sigmoid(
            jnp.dot(s1.astype(jnp.bfloat16), rw_ref[...],
                    preferred_element_type=jnp.float32) + rb_ref[...])
        h_t = s2 * (conv + cb_ref[...])
        hcen = hseq_ref[pl.ds(t0 + PAD, TB)].astype(jnp.float32)
        rnn = h_t.reshape(TB, B, HIDDEN) + hcen

        tvec = t0 + lax.broadcasted_iota(jnp.int32, (TB, B, 1), 0)
        m = (tvec == li.reshape(1, B, 1)).astype(jnp.float32)
        acc_ref[...] += jnp.sum(rnn * m, axis=0)
        return 0

    jax.lax.fori_loop(0, NT, win_block, 0)
    out_ref[...] = (jnp.dot(acc_ref[...], fw_ref[...],
                            preferred_element_type=jnp.float32) + fb_ref[...])


@jax.jit
def kernel(batchdata, emb_table, kernel_W, kernel_b, rec_W, rec_b, scale_W,
           scale_b, rescale_W, rescale_b, conv_W, conv_b, fc_W, fc_b):
    wkm = jnp.zeros((D, 128), jnp.float32).at[:, 0:6].set(kernel_W[0:6, 0:D].T)
    wkr = kernel_W[6:, 0:D].T
    wm = jnp.zeros((HIDDEN, 128), jnp.float32).at[:, 0:6].set(
        rec_W[0:6, 0:HIDDEN].T)
    wrr = rec_W[6:, 0:HIDDEN].T
    bias_full = kernel_b + kernel_W[:, D] + rec_b + rec_W[:, HIDDEN]
    bm = jnp.zeros((1, 128), jnp.float32).at[0, 0:6].set(bias_full[0:6])
    br = bias_full[6:].reshape(1, GATE_REST)
    wc = jnp.transpose(conv_W, (2, 1, 0)).reshape(CONV * HIDDEN, HIDDEN)
    sw = scale_W.T
    sb = scale_b.reshape(1, -1)
    rw = rescale_W.T
    rb = rescale_b.reshape(1, -1)
    cb = conv_b.reshape(1, -1)
    fw = fc_W.T
    fb = fc_b.reshape(1, -1)

    bf = jnp.bfloat16
    full = lambda shape: pl.BlockSpec(shape, lambda: tuple(0 for _ in shape))
    gfull = lambda shape: pl.BlockSpec(shape,
                                       lambda i: tuple(0 for _ in shape))
    x, cnt = pl.pallas_call(
        _emb_body,
        grid=(B,),
        in_specs=[
            pl.BlockSpec((1, T, V), lambda b: (b, 0, 0)),
            gfull((V, D)),
        ],
        out_specs=[
            pl.BlockSpec((1, T, D), lambda b: (b, 0, 0)),
            pl.BlockSpec((1, 8, 128), lambda b: (b, 0, 0)),
        ],
        out_shape=[
            jax.ShapeDtypeStruct((B, T, D), jnp.float32),
            jax.ShapeDtypeStruct((B, 8, 128), jnp.float32),
        ],
    )(batchdata, emb_table.astype(bf))
    xT = jnp.transpose(x, (1, 0, 2))
    li = jnp.clip(cnt[:, 0, 0].astype(jnp.int32) - 1, 0, T - 1).reshape(B, 1)

    x1m, x1r = pl.pallas_call(
        _x1_body,
        grid=(NT,),
        in_specs=[
            pl.BlockSpec((TB, B, D), lambda i: (i, 0, 0)),
            gfull((D, 128)),
            gfull((D, GATE_REST)),
            gfull((1, 128)),
            gfull((1, GATE_REST)),
        ],
        out_specs=[
            pl.BlockSpec((TB, B, 128), lambda i: (i, 0, 0)),
            pl.BlockSpec((TB, B, GATE_REST), lambda i: (i, 0, 0)),
        ],
        out_shape=[
            jax.ShapeDtypeStruct((T, B, 128), jnp.float32),
            jax.ShapeDtypeStruct((T, B, GATE_REST), jnp.bfloat16),
        ],
    )(xT, wkm.astype(bf), wkr.astype(bf), bm, br)

    rec_args = (x1m, x1r, wm.astype(bf), wrr.astype(bf), li, wc.astype(bf),
                sw.astype(bf), sb, rw.astype(bf), rb, cb, fw, fb)
    logits = pl.pallas_call(
        _rec_body,
        in_specs=[full(a.shape) for a in rec_args],
        out_specs=full((B, OUT_DIM)),
        out_shape=jax.ShapeDtypeStruct((B, OUT_DIM), jnp.float32),
        scratch_shapes=[
            pltpu.VMEM((B, HIDDEN), jnp.float32),
            pltpu.VMEM((B, HIDDEN), jnp.float32),
            pltpu.VMEM((B, 128), jnp.float32),
            pltpu.VMEM((PAD + T, B, HIDDEN), jnp.bfloat16),
            pltpu.VMEM((T, B, 128), jnp.float32),
            pltpu.VMEM((B, HIDDEN), jnp.float32),
        ],
    )(*rec_args)
    return logits


# speedup vs baseline: 1.2181x; 1.1870x over previous
import jax
import jax.numpy as jnp
from jax import lax
from jax.experimental import pallas as pl
from jax.experimental.pallas import tpu as pltpu

B, T, V = 16, 512, 2048
D = 128
LEVELS = 3
CHUNK = 128
HIDDEN = CHUNK * LEVELS
CONV = 10
OUT_DIM = 128
GATE_REST = 4 * LEVELS * CHUNK
XH = D + HIDDEN
PAD = 16
TB = 64
NT = T // TB


def _emb_body(bd_ref, emb_ref, x_ref, cnt_ref):
    bd = bd_ref[0]
    xf = (bd == 1).astype(jnp.bfloat16)
    y = jnp.dot(xf, emb_ref[...], preferred_element_type=jnp.float32)
    x_ref[0] = y
    m = jnp.max(jnp.abs(y), axis=1, keepdims=True)
    cnt = jnp.sum((m > 0.0).astype(jnp.float32))
    cnt_ref[...] = jnp.full((1, 8, 128), cnt, jnp.float32)


def _rec_body(x_ref, wm_ref, wr_ref, bm_ref, br_ref, hseq_ref, dis_ref,
              c_ref, h_ref, d_ref):
    wm = wm_ref[...]
    wr = wr_ref[...]
    bm = bm_ref[...]
    br = br_ref[...]

    c_ref[...] = jnp.zeros((B, HIDDEN), jnp.float32)
    h_ref[...] = jnp.zeros((B, HIDDEN), jnp.float32)
    d_ref[...] = jnp.zeros((B, 128), jnp.float32)
    hseq_ref[0:PAD] = jnp.zeros((PAD, B, HIDDEN), jnp.bfloat16)

    def one_step(t, c, h, dbuf):
        xt = x_ref[t]

        xh = jnp.concatenate([xt, h], axis=1).astype(jnp.bfloat16)
        xom = jnp.dot(xh, wm, preferred_element_type=jnp.float32) + bm
        xor_ = jnp.dot(xh, wr, preferred_element_type=jnp.float32) + br

        f_in = xom[:, 0:3]
        i_in = xom[:, 3:6]
        fe = jnp.exp(f_in - jnp.max(f_in, axis=1, keepdims=True))
        fp = fe / jnp.sum(fe, axis=1, keepdims=True)
        p0, p1, p2 = fp[:, 0:1], fp[:, 1:2], fp[:, 2:3]
        fm = (p0, p0 + p1, (p0 + p1) + p2)
        ie = jnp.exp(i_in - jnp.max(i_in, axis=1, keepdims=True))
        ip = ie / jnp.sum(ie, axis=1, keepdims=True)
        q0, q1, q2 = ip[:, 0:1], ip[:, 1:2], ip[:, 2:3]
        im = ((q2 + q1) + q0, q2 + q1, q2)

        c_parts = []
        h_parts = []
        for l in range(LEVELS):
            fg = jax.nn.sigmoid(xor_[:, l * CHUNK:(l + 1) * CHUNK])
            ig = jax.nn.sigmoid(xor_[:, (3 + l) * CHUNK:(4 + l) * CHUNK])
            og = jax.nn.sigmoid(xor_[:, (6 + l) * CHUNK:(7 + l) * CHUNK])
            ci = jnp.tanh(xor_[:, (9 + l) * CHUNK:(10 + l) * CHUNK])
            cl = c[:, l * CHUNK:(l + 1) * CHUNK]
            ov = fm[l] * im[l]
            c3 = cl * (ov * (fg - 1.0) + fm[l]) + ci * (ov * (ig - 1.0)
                                                        + im[l])
            h_parts.append(og * jnp.tanh(c3))
            c_parts.append(c3)
        c_new = jnp.concatenate(c_parts, axis=1)
        h_new = jnp.concatenate(h_parts, axis=1)

        cur_dis = 1.0 - (fm[0] + fm[1] + fm[2]) * (1.0 / 3.0)
        dnew = jnp.concatenate(
            [dbuf[:, 1:10], cur_dis, dbuf[:, 10:128]], axis=1)

        run = dnew[:, 0:1]
        cs = [run]
        for k in range(1, CONV):
            run = run + dnew[:, k:k + 1]
            cs.append(run)
        mx = cs[0]
        for k in range(1, CONV):
            mx = jnp.maximum(mx, cs[k])
        es = [jnp.exp(v - mx) for v in cs]
        tot = es[0]
        for k in range(1, CONV):
            tot = tot + es[k]
        inv = 1.0 / tot
        dn = jnp.concatenate([e * inv for e in es]
                             + [jnp.zeros((B, 128 - CONV), jnp.float32)],
                             axis=1)

        hseq_ref[PAD + t] = h_new.astype(jnp.bfloat16)
        dis_ref[t] = dn
        return c_new, h_new, dnew

    def step2(i, _):
        c = c_ref[...]
        h = h_ref[...]
        dbuf = d_ref[...]
        t = 2 * i
        c, h, dbuf = one_step(t, c, h, dbuf)
        c, h, dbuf = one_step(t + 1, c, h, dbuf)
        c_ref[...] = c
        h_ref[...] = h
        d_ref[...] = dbuf
        return 0

    jax.lax.fori_loop(0, T // 2, step2, 0)


def _win_body(hseq_ref, dis_ref, li_ref, wc_ref, sw_ref, sb_ref, rw_ref,
              rb_ref, cb_ref, fw_ref, fb_ref, out_ref, acc_ref):
    tb = pl.program_id(0)
    t0 = tb * TB

    @pl.when(tb == 0)
    def _init():
        acc_ref[...] = jnp.zeros((B, HIDDEN), jnp.float32)

    theme = None
    conv = None
    for k in range(CONV):
        hk = hseq_ref[pl.ds(t0 + PAD - (CONV - 1) + k, TB)]
        dk = dis_ref[pl.ds(t0, TB), :, k:k + 1]
        shk = (hk.astype(jnp.float32).reshape(TB * B, HIDDEN)
               * dk.reshape(TB * B, 1))
        theme = shk if theme is None else theme + shk
        pk = jnp.dot(shk.astype(jnp.bfloat16),
                     wc_ref[k * HIDDEN:(k + 1) * HIDDEN],
                     preferred_element_type=jnp.float32)
        conv = pk if conv is None else conv + pk
    s1 = jnp.maximum(
        jnp.dot((theme * (1.0 / CONV)).astype(jnp.bfloat16), sw_ref[...],
                preferred_element_type=jnp.float32) + sb_ref[...], 0.0)
    s2 = jax.nn.sigmoid(
        jnp.dot(s1.astype(jnp.bfloat16), rw_ref[...],
                preferred_element_type=jnp.float32) + rb_ref[...])
    h_t = s2 * (conv + cb_ref[...])
    hcen = hseq_ref[pl.ds(t0 + PAD, TB)].astype(jnp.float32)
    rnn = h_t.reshape(TB, B, HIDDEN) + hcen

    tvec = t0 + lax.broadcasted_iota(jnp.int32, (TB, B, 1), 0)
    m = (tvec == li_ref[...].reshape(1, B, 1)).astype(jnp.float32)
    acc_ref[...] += jnp.sum(rnn * m, axis=0)

    @pl.when(tb == NT - 1)
    def _fin():
        out_ref[...] = (jnp.dot(acc_ref[...], fw_ref[...],
                                preferred_element_type=jnp.float32)
                        + fb_ref[...])


@jax.jit
def kernel(batchdata, emb_table, kernel_W, kernel_b, rec_W, rec_b, scale_W,
           scale_b, rescale_W, rescale_b, conv_W, conv_b, fc_W, fc_b):
    x, cnt = pl.pallas_call(
        _emb_body,
        grid=(B,),
        in_specs=[
            pl.BlockSpec((1, T, V), lambda b: (b, 0, 0)),
            pl.BlockSpec((V, D), lambda b: (0, 0)),
        ],
        out_specs=[
            pl.BlockSpec((1, T, D), lambda b: (b, 0, 0)),
            pl.BlockSpec((1, 8, 128), lambda b: (b, 0, 0)),
        ],
        out_shape=[
            jax.ShapeDtypeStruct((B, T, D), jnp.float32),
            jax.ShapeDtypeStruct((B, 8, 128), jnp.float32),
        ],
    )(batchdata, emb_table.astype(jnp.bfloat16))

    xT = jnp.transpose(x, (1, 0, 2))
    li = jnp.clip(cnt[:, 0, 0].astype(jnp.int32) - 1, 0, T - 1).reshape(B, 1)

    wxm = kernel_W[0:6, 0:D].T
    whm = rec_W[0:6, 0:HIDDEN].T
    wm = jnp.zeros((XH, 128), jnp.float32)
    wm = wm.at[0:D, 0:6].set(wxm).at[D:XH, 0:6].set(whm)
    wr = jnp.concatenate([kernel_W[6:, 0:D].T, rec_W[6:, 0:HIDDEN].T],
                         axis=0)
    bias_full = kernel_b + kernel_W[:, D] + rec_b + rec_W[:, HIDDEN]
    bm = jnp.zeros((1, 128), jnp.float32).at[0, 0:6].set(bias_full[0:6])
    br = bias_full[6:].reshape(1, GATE_REST)
    wc = jnp.transpose(conv_W, (2, 1, 0)).reshape(CONV * HIDDEN, HIDDEN)
    sw = scale_W.T
    sb = scale_b.reshape(1, -1)
    rw = rescale_W.T
    rb = rescale_b.reshape(1, -1)
    cb = conv_b.reshape(1, -1)
    fw = fc_W.T
    fb = fc_b.reshape(1, -1)

    bf = jnp.bfloat16
    full = lambda shape: pl.BlockSpec(shape, lambda: tuple(0 for _ in shape))
    rec_args = (xT, wm.astype(bf), wr.astype(bf), bm, br)
    hseq, dis = pl.pallas_call(
        _rec_body,
        in_specs=[full(a.shape) for a in rec_args],
        out_specs=[full((PAD + T, B, HIDDEN)), full((T, B, 128))],
        out_shape=[
            jax.ShapeDtypeStruct((PAD + T, B, HIDDEN), jnp.bfloat16),
            jax.ShapeDtypeStruct((T, B, 128), jnp.float32),
        ],
        scratch_shapes=[
            pltpu.VMEM((B, HIDDEN), jnp.float32),
            pltpu.VMEM((B, HIDDEN), jnp.float32),
            pltpu.VMEM((B, 128), jnp.float32),
        ],
    )(*rec_args)

    win_args = (hseq, dis, li, wc.astype(bf), sw.astype(bf), sb,
                rw.astype(bf), rb, cb, fw, fb)
    gfull = lambda shape: pl.BlockSpec(shape,
                                       lambda i: tuple(0 for _ in shape))
    logits = pl.pallas_call(
        _win_body,
        grid=(NT,),
        in_specs=[gfull(a.shape) for a in win_args],
        out_specs=gfull((B, OUT_DIM)),
        out_shape=jax.ShapeDtypeStruct((B, OUT_DIM), jnp.float32),
        scratch_shapes=[pltpu.VMEM((B, HIDDEN), jnp.float32)],
    )(*win_args)
    return logits


# speedup vs baseline: 1.2245x; 1.0052x over previous
import jax
import jax.numpy as jnp
from jax import lax
from jax.experimental import pallas as pl
from jax.experimental.pallas import tpu as pltpu

B, T, V = 16, 512, 2048
D = 128
LEVELS = 3
CHUNK = 128
HIDDEN = CHUNK * LEVELS
CONV = 10
OUT_DIM = 128
GATE_REST = 4 * LEVELS * CHUNK
XH = D + HIDDEN
PAD = 16
TB = 64
NT = T // TB


def _emb_body(bd_ref, emb_ref, x_ref, cnt_ref):
    bd = bd_ref[0]
    xf = (bd == 1).astype(jnp.bfloat16)
    y = jnp.dot(xf, emb_ref[...], preferred_element_type=jnp.float32)
    x_ref[0] = y.astype(jnp.bfloat16)
    m = jnp.max(jnp.abs(y), axis=1, keepdims=True)
    cnt = jnp.sum((m > 0.0).astype(jnp.float32))
    cnt_ref[...] = jnp.full((1, 8, 128), cnt, jnp.float32)


def _rec_body(x_ref, wm_ref, wr_ref, bm_ref, br_ref, hseq_ref, dis_ref,
              c_ref, h_ref, d_ref):
    wm = wm_ref[...]
    wr = wr_ref[...]
    bm = bm_ref[...]
    br = br_ref[...]

    c_ref[...] = jnp.zeros((B, HIDDEN), jnp.float32)
    h_ref[...] = jnp.zeros((B, HIDDEN), jnp.float32)
    d_ref[...] = jnp.zeros((B, 128), jnp.float32)
    hseq_ref[0:PAD] = jnp.zeros((PAD, B, HIDDEN), jnp.bfloat16)

    def one_step(t, c, h, dbuf):
        xt = x_ref[t]

        xh = jnp.concatenate([xt, h.astype(jnp.bfloat16)], axis=1)
        xom = jnp.dot(xh, wm, preferred_element_type=jnp.float32) + bm
        xor_ = jnp.dot(xh, wr, preferred_element_type=jnp.float32) + br

        f_in = xom[:, 0:3]
        i_in = xom[:, 3:6]
        fe = jnp.exp(f_in - jnp.max(f_in, axis=1, keepdims=True))
        fp = fe / jnp.sum(fe, axis=1, keepdims=True)
        p0, p1, p2 = fp[:, 0:1], fp[:, 1:2], fp[:, 2:3]
        fm = (p0, p0 + p1, (p0 + p1) + p2)
        ie = jnp.exp(i_in - jnp.max(i_in, axis=1, keepdims=True))
        ip = ie / jnp.sum(ie, axis=1, keepdims=True)
        q0, q1, q2 = ip[:, 0:1], ip[:, 1:2], ip[:, 2:3]
        im = ((q2 + q1) + q0, q2 + q1, q2)

        c_parts = []
        h_parts = []
        for l in range(LEVELS):
            fg = jax.nn.sigmoid(xor_[:, l * CHUNK:(l + 1) * CHUNK])
            ig = jax.nn.sigmoid(xor_[:, (3 + l) * CHUNK:(4 + l) * CHUNK])
            og = jax.nn.sigmoid(xor_[:, (6 + l) * CHUNK:(7 + l) * CHUNK])
            ci = jnp.tanh(xor_[:, (9 + l) * CHUNK:(10 + l) * CHUNK])
            cl = c[:, l * CHUNK:(l + 1) * CHUNK]
            ov = fm[l] * im[l]
            c3 = cl * (ov * (fg - 1.0) + fm[l]) + ci * (ov * (ig - 1.0)
                                                        + im[l])
            h_parts.append(og * jnp.tanh(c3))
            c_parts.append(c3)
        c_new = jnp.concatenate(c_parts, axis=1)
        h_new = jnp.concatenate(h_parts, axis=1)

        cur_dis = 1.0 - (fm[0] + fm[1] + fm[2]) * (1.0 / 3.0)
        dnew = jnp.concatenate(
            [dbuf[:, 1:10], cur_dis, dbuf[:, 10:128]], axis=1)

        run = dnew[:, 0:1]
        cs = [run]
        for k in range(1, CONV):
            run = run + dnew[:, k:k + 1]
            cs.append(run)
        mx = cs[0]
        for k in range(1, CONV):
            mx = jnp.maximum(mx, cs[k])
        es = [jnp.exp(v - mx) for v in cs]
        tot = es[0]
        for k in range(1, CONV):
            tot = tot + es[k]
        inv = 1.0 / tot
        dn = jnp.concatenate([e * inv for e in es]
                             + [jnp.zeros((B, 128 - CONV), jnp.float32)],
                             axis=1)

        hseq_ref[PAD + t] = h_new.astype(jnp.bfloat16)
        dis_ref[t] = dn
        return c_new, h_new, dnew

    def step2(i, _):
        c = c_ref[...]
        h = h_ref[...]
        dbuf = d_ref[...]
        t = 2 * i
        c, h, dbuf = one_step(t, c, h, dbuf)
        c, h, dbuf = one_step(t + 1, c, h, dbuf)
        c_ref[...] = c
        h_ref[...] = h
        d_ref[...] = dbuf
        return 0

    jax.lax.fori_loop(0, T // 2, step2, 0)


def _win_body(hseq_ref, dis_ref, li_ref, wc_ref, sw_ref, sb_ref, rw_ref,
              rb_ref, cb_ref, fw_ref, fb_ref, out_ref, acc_ref):
    tb = pl.program_id(0)
    t0 = tb * TB

    @pl.when(tb == 0)
    def _init():
        acc_ref[...] = jnp.zeros((B, HIDDEN), jnp.float32)

    theme = None
    conv = None
    for k in range(CONV):
        hk = hseq_ref[pl.ds(t0 + PAD - (CONV - 1) + k, TB)]
        dk = dis_ref[pl.ds(t0, TB), :, k:k + 1]
        shk = (hk.astype(jnp.float32).reshape(TB * B, HIDDEN)
               * dk.reshape(TB * B, 1))
        theme = shk if theme is None else theme + shk
        pk = jnp.dot(shk.astype(jnp.bfloat16),
                     wc_ref[k * HIDDEN:(k + 1) * HIDDEN],
                     preferred_element_type=jnp.float32)
        conv = pk if conv is None else conv + pk
    s1 = jnp.maximum(
        jnp.dot((theme * (1.0 / CONV)).astype(jnp.bfloat16), sw_ref[...],
                preferred_element_type=jnp.float32) + sb_ref[...], 0.0)
    s2 = jax.nn.sigmoid(
        jnp.dot(s1.astype(jnp.bfloat16), rw_ref[...],
                preferred_element_type=jnp.float32) + rb_ref[...])
    h_t = s2 * (conv + cb_ref[...])
    hcen = hseq_ref[pl.ds(t0 + PAD, TB)].astype(jnp.float32)
    rnn = h_t.reshape(TB, B, HIDDEN) + hcen

    tvec = t0 + lax.broadcasted_iota(jnp.int32, (TB, B, 1), 0)
    m = (tvec == li_ref[...].reshape(1, B, 1)).astype(jnp.float32)
    acc_ref[...] += jnp.sum(rnn * m, axis=0)

    @pl.when(tb == NT - 1)
    def _fin():
        out_ref[...] = (jnp.dot(acc_ref[...], fw_ref[...],
                                preferred_element_type=jnp.float32)
                        + fb_ref[...])


@jax.jit
def kernel(batchdata, emb_table, kernel_W, kernel_b, rec_W, rec_b, scale_W,
           scale_b, rescale_W, rescale_b, conv_W, conv_b, fc_W, fc_b):
    x, cnt = pl.pallas_call(
        _emb_body,
        grid=(B,),
        in_specs=[
            pl.BlockSpec((1, T, V), lambda b: (b, 0, 0)),
            pl.BlockSpec((V, D), lambda b: (0, 0)),
        ],
        out_specs=[
            pl.BlockSpec((1, T, D), lambda b: (b, 0, 0)),
            pl.BlockSpec((1, 8, 128), lambda b: (b, 0, 0)),
        ],
        out_shape=[
            jax.ShapeDtypeStruct((B, T, D), jnp.bfloat16),
            jax.ShapeDtypeStruct((B, 8, 128), jnp.float32),
        ],
    )(batchdata, emb_table.astype(jnp.bfloat16))

    xT = jnp.transpose(x, (1, 0, 2))
    li = jnp.clip(cnt[:, 0, 0].astype(jnp.int32) - 1, 0, T - 1).reshape(B, 1)

    wxm = kernel_W[0:6, 0:D].T
    whm = rec_W[0:6, 0:HIDDEN].T
    wm = jnp.zeros((XH, 128), jnp.float32)
    wm = wm.at[0:D, 0:6].set(wxm).at[D:XH, 0:6].set(whm)
    wr = jnp.concatenate([kernel_W[6:, 0:D].T, rec_W[6:, 0:HIDDEN].T],
                         axis=0)
    bias_full = kernel_b + kernel_W[:, D] + rec_b + rec_W[:, HIDDEN]
    bm = jnp.zeros((1, 128), jnp.float32).at[0, 0:6].set(bias_full[0:6])
    br = bias_full[6:].reshape(1, GATE_REST)
    wc = jnp.transpose(conv_W, (2, 1, 0)).reshape(CONV * HIDDEN, HIDDEN)
    sw = scale_W.T
    sb = scale_b.reshape(1, -1)
    rw = rescale_W.T
    rb = rescale_b.reshape(1, -1)
    cb = conv_b.reshape(1, -1)
    fw = fc_W.T
    fb = fc_b.reshape(1, -1)

    bf = jnp.bfloat16
    full = lambda shape: pl.BlockSpec(shape, lambda: tuple(0 for _ in shape))
    rec_args = (xT, wm.astype(bf), wr.astype(bf), bm, br)
    hseq, dis = pl.pallas_call(
        _rec_body,
        in_specs=[full(a.shape) for a in rec_args],
        out_specs=[full((PAD + T, B, HIDDEN)), full((T, B, 128))],
        out_shape=[
            jax.ShapeDtypeStruct((PAD + T, B, HIDDEN), jnp.bfloat16),
            jax.ShapeDtypeStruct((T, B, 128), jnp.float32),
        ],
        scratch_shapes=[
            pltpu.VMEM((B, HIDDEN), jnp.float32),
            pltpu.VMEM((B, HIDDEN), jnp.float32),
            pltpu.VMEM((B, 128), jnp.float32),
        ],
    )(*rec_args)

    win_args = (hseq, dis, li, wc.astype(bf), sw.astype(bf), sb,
                rw.astype(bf), rb, cb, fw, fb)
    gfull = lambda shape: pl.BlockSpec(shape,
                                       lambda i: tuple(0 for _ in shape))
    logits = pl.pallas_call(
        _win_body,
        grid=(NT,),
        in_specs=[gfull(a.shape) for a in win_args],
        out_specs=gfull((B, OUT_DIM)),
        out_shape=jax.ShapeDtypeStruct((B, OUT_DIM), jnp.float32),
        scratch_shapes=[pltpu.VMEM((B, HIDDEN), jnp.float32)],
    )(*win_args)
    return logits
